# Initial kernel scaffold; baseline (speedup 1.0000x reference)
import jax
import jax.numpy as jnp
from jax.experimental import pallas as pl


def kernel(stacks, mask, W1, b1, W2, b2, W3, b3, Wc1, bc1, Wc2, bc2):
    raise NotImplementedError("write your pallas kernel here")


# speedup vs baseline: 1.0363x; 1.0363x over previous
import jax
import jax.numpy as jnp
from jax.experimental import pallas as pl
from jax.experimental.pallas import tpu as pltpu


def _mlp_block(x_ref, m_ref, W1r, b1r, W2r, b2r, W3r, b3r, Wc1r, bc1r,
               Wc2r, bc2r, out_ref):
    nh = x_ref.shape[1]
    r = x_ref.shape[2]
    hc_dim = Wc1r.shape[1]
    acc = jnp.zeros((r, hc_dim), jnp.float32)
    for i in range(nh):
        x = x_ref[0, i].astype(jnp.bfloat16)
        h = jnp.dot(x, W1r[i].astype(jnp.bfloat16),
                    preferred_element_type=jnp.float32) + b1r[i]
        h = jnp.maximum(h, 0.0).astype(jnp.bfloat16)
        h = jnp.dot(h, W2r[i].astype(jnp.bfloat16),
                    preferred_element_type=jnp.float32) + b2r[i]
        h = jnp.maximum(h, 0.0).astype(jnp.bfloat16)
        w3c = (W3r[i] * Wc1r[i:i + 1, :]).astype(jnp.bfloat16)
        acc = acc + jnp.dot(h, w3c, preferred_element_type=jnp.float32)
    bfuse = bc1r[0] + jnp.sum(b3r[...] * Wc1r[...], axis=0)
    hc = jnp.maximum(acc + bfuse[None, :], 0.0)
    oc = jnp.dot(hc, Wc2r[...], preferred_element_type=jnp.float32) + bc2r[0]
    out_ref[0] = oc * m_ref[0]


def kernel(stacks, mask, W1, b1, W2, b2, W3, b3, Wc1, bc1, Wc2, bc2):
    B, NH, N, _, D = stacks.shape
    H = W1.shape[-1]
    HC = Wc1.shape[-1]
    DOUT = Wc2.shape[-1]
    NN = N * N

    R = NN
    for cand in (2048, 1024, 512, 256, 128, 64, 32, 16, 8):
        if NN % cand == 0:
            R = cand
            break

    xs = stacks.reshape(B, NH, NN, D)
    mf = mask.astype(jnp.float32).reshape(B, NN, 1)
    bc1r = bc1.reshape(1, HC)
    bc2r = bc2.reshape(1, DOUT)

    grid = (B, NN // R)
    full = lambda shape: pl.BlockSpec(shape, lambda b, j: (0,) * len(shape))
    out = pl.pallas_call(
        _mlp_block,
        grid=grid,
        in_specs=[
            pl.BlockSpec((1, NH, R, D), lambda b, j: (b, 0, j, 0)),
            pl.BlockSpec((1, R, 1), lambda b, j: (b, j, 0)),
            full(W1.shape), full(b1.shape), full(W2.shape), full(b2.shape),
            full(W3.shape), full(b3.shape), full(Wc1.shape), full(bc1r.shape),
            full(Wc2.shape), full(bc2r.shape),
        ],
        out_specs=pl.BlockSpec((1, R, DOUT), lambda b, j: (b, j, 0)),
        out_shape=jax.ShapeDtypeStruct((B, NN, DOUT), jnp.float32),
        compiler_params=pltpu.CompilerParams(
            dimension_semantics=("parallel", "parallel")),
    )(xs, mf, W1, b1, W2, b2, W3, b3, Wc1, bc1r, Wc2, bc2r)
    return out.reshape(B, N, N, DOUT)


# speedup vs baseline: 1.0479x; 1.0111x over previous
import jax
import jax.numpy as jnp
from jax.experimental import pallas as pl
from jax.experimental.pallas import tpu as pltpu


def _mlp_block(x_ref, m_ref, W1r, b1r, W2r, b2r, W3sr, bfr, Wc2r, bc2r,
               out_ref):
    nh = x_ref.shape[1]
    r = x_ref.shape[2]
    h2s = []
    for p in range(nh // 2):
        h1s = []
        for q in (2 * p, 2 * p + 1):
            x = x_ref[0, q].astype(jnp.bfloat16)
            h1 = jnp.dot(x, W1r[q], preferred_element_type=jnp.float32)
            h1 = jnp.maximum(h1 + b1r[q], 0.0).astype(jnp.bfloat16)
            h1s.append(h1)
        h1pair = jnp.concatenate(h1s, axis=-1)
        h2 = jnp.dot(h1pair, W2r[p], preferred_element_type=jnp.float32)
        h2 = jnp.maximum(h2 + b2r[p], 0.0).astype(jnp.bfloat16)
        h2s.append(h2)
    h2all = jnp.concatenate(h2s, axis=-1)
    acc = jnp.dot(h2all, W3sr[...], preferred_element_type=jnp.float32)
    hc = jnp.maximum(acc + bfr[0], 0.0).astype(jnp.bfloat16)
    oc = jnp.dot(hc, Wc2r[...], preferred_element_type=jnp.float32) + bc2r[0]
    out_ref[0] = oc * m_ref[0]


def kernel(stacks, mask, W1, b1, W2, b2, W3, b3, Wc1, bc1, Wc2, bc2):
    B, NH, N, _, D = stacks.shape
    H = W1.shape[-1]
    HC = Wc1.shape[-1]
    DOUT = Wc2.shape[-1]
    NN = N * N
    NP = NH // 2

    R = NN
    for cand in (2048, 1024, 512, 256, 128, 64, 32, 16, 8):
        if NN % cand == 0:
            R = cand
            break

    xs = stacks.reshape(B, NH, NN, D)
    mf = mask.astype(jnp.float32).reshape(B, NN, 1)

    z = jnp.zeros((NP, H, H), jnp.float32)
    W2bd = jnp.concatenate([
        jnp.concatenate([W2[0::2], z], axis=2),
        jnp.concatenate([z, W2[1::2]], axis=2),
    ], axis=1).astype(jnp.bfloat16)
    b2p = b2.reshape(NP, 2 * H)
    W3s = (W3 * Wc1[:, None, :]).reshape(NH * H, HC).astype(jnp.bfloat16)
    bf = (bc1 + jnp.sum(b3 * Wc1, axis=0)).reshape(1, HC)
    W1b = W1.astype(jnp.bfloat16)
    Wc2b = Wc2.astype(jnp.bfloat16)
    bc2r = bc2.reshape(1, DOUT)

    grid = (B, NN // R)
    full = lambda shape: pl.BlockSpec(shape, lambda b, j: (0,) * len(shape))
    out = pl.pallas_call(
        _mlp_block,
        grid=grid,
        in_specs=[
            pl.BlockSpec((1, NH, R, D), lambda b, j: (b, 0, j, 0)),
            pl.BlockSpec((1, R, 1), lambda b, j: (b, j, 0)),
            full(W1b.shape), full(b1.shape), full(W2bd.shape), full(b2p.shape),
            full(W3s.shape), full(bf.shape), full(Wc2b.shape),
            full(bc2r.shape),
        ],
        out_specs=pl.BlockSpec((1, R, DOUT), lambda b, j: (b, j, 0)),
        out_shape=jax.ShapeDtypeStruct((B, NN, DOUT), jnp.float32),
        compiler_params=pltpu.CompilerParams(
            dimension_semantics=("parallel", "parallel")),
    )(xs, mf, W1b, b1, W2bd, b2p, W3s, bf, Wc2b, bc2r)
    return out.reshape(B, N, N, DOUT)
